# baseline (speedup 1.0000x reference)
import jax
import jax.numpy as jnp
from jax import lax
from jax.experimental import pallas as pl
from jax.experimental.pallas import tpu as pltpu
from jax.experimental.pallas import tpu_sc as plsc

B, L, D = 1024, 20, 64
HOURS = 24 * 7
NC, NS = 2, 16
NW = NC * NS
ROWS = B * L
RPW = ROWS // NW
BPW = B // NW
GCHUNK = 128
NCHUNK = RPW // GCHUNK
Q = L * L // 2
SPW = BPW * Q
HALF = SPW // 2


def _sc_body(wt_hbm, wl_hbm, wu_hbm, traj_hbm, mat_hbm,
             out_hbm, dse_hbm, dte_hbm, dso_hbm, dto_hbm,
             traj_v, uidx_v, lidx_v, tidx_v, acc_v, tmp_v,
             mat_v, se_v, te_v, so_v, to_v, sem):
    wid = lax.axis_index("s") * NC + lax.axis_index("c")
    base = wid * RPW
    pltpu.sync_copy(traj_hbm.at[pl.ds(base * 3, RPW * 3)], traj_v)

    def split_idx(k, _):
        off = 48 * k + 3 * lax.iota(jnp.int32, 16)
        sl = pl.ds(k * 16, 16)
        uidx_v[sl] = plsc.load_gather(traj_v, [off])
        lidx_v[sl] = plsc.load_gather(traj_v, [off + 1])
        t = plsc.load_gather(traj_v, [off + 2])
        tidx_v[sl] = (t + (HOURS - 1)) % HOURS + 1
        return 0

    lax.fori_loop(0, RPW // 16, split_idx, 0, unroll=4)

    cps = []
    for idx_v, table, dst_v in ((uidx_v, wu_hbm, acc_v),
                                (lidx_v, wl_hbm, tmp_v)):
        for k in range(NCHUNK):
            cps.append(pltpu.async_copy(
                table.at[idx_v.at[pl.ds(k * GCHUNK, GCHUNK)]],
                dst_v.at[pl.ds(k * GCHUNK, GCHUNK), :], sem))

    def split_mat(k, _):
        off = 64 * k + 4 * lax.iota(jnp.int32, 16)
        sl = pl.ds(k * 16, 16)
        se_v[sl] = plsc.load_gather(mat_v, [off])
        te_v[sl] = plsc.load_gather(mat_v, [off + 1])
        so_v[sl] = plsc.load_gather(mat_v, [off + 2])
        to_v[sl] = plsc.load_gather(mat_v, [off + 3])
        return 0

    for h in range(2):
        pltpu.sync_copy(
            mat_hbm.at[pl.ds((wid * SPW + h * HALF) * 4, HALF * 4)], mat_v)
        lax.fori_loop(0, HALF // 16, split_mat, 0, unroll=4)
        sbase = wid * SPW + h * HALF
        pltpu.sync_copy(se_v, dse_hbm.at[pl.ds(sbase, HALF)])
        pltpu.sync_copy(te_v, dte_hbm.at[pl.ds(sbase, HALF)])
        pltpu.sync_copy(so_v, dso_hbm.at[pl.ds(sbase, HALF)])
        pltpu.sync_copy(to_v, dto_hbm.at[pl.ds(sbase, HALF)])

    for cp in cps:
        cp.wait()

    def addrow(r, _):
        for c in range(D // 16):
            sl = pl.ds(c * 16, 16)
            acc_v[r, sl] = acc_v[r, sl] + tmp_v[r, sl]
        return 0

    lax.fori_loop(0, RPW, addrow, 0, unroll=4)

    cps = []
    for k in range(NCHUNK):
        cps.append(pltpu.async_copy(
            wt_hbm.at[tidx_v.at[pl.ds(k * GCHUNK, GCHUNK)]],
            tmp_v.at[pl.ds(k * GCHUNK, GCHUNK), :], sem))
    for cp in cps:
        cp.wait()
    lax.fori_loop(0, RPW, addrow, 0, unroll=4)
    pltpu.sync_copy(acc_v, out_hbm.at[pl.ds(base, RPW)])


def _sc_joint_and_streams(W_t, W_l, W_u, traj_flat, mat_flat):
    mesh = plsc.VectorSubcoreMesh(core_axis_name="c", subcore_axis_name="s")
    return pl.kernel(
        _sc_body,
        out_type=(
            jax.ShapeDtypeStruct((ROWS, D), jnp.float32),
            jax.ShapeDtypeStruct((B * Q,), jnp.float32),
            jax.ShapeDtypeStruct((B * Q,), jnp.float32),
            jax.ShapeDtypeStruct((B * Q,), jnp.float32),
            jax.ShapeDtypeStruct((B * Q,), jnp.float32),
        ),
        mesh=mesh,
        scratch_types=[
            pltpu.VMEM((RPW * 3,), jnp.int32),
            pltpu.VMEM((RPW,), jnp.int32),
            pltpu.VMEM((RPW,), jnp.int32),
            pltpu.VMEM((RPW,), jnp.int32),
            pltpu.VMEM((RPW, D), jnp.float32),
            pltpu.VMEM((RPW, D), jnp.float32),
            pltpu.VMEM((HALF * 4,), jnp.float32),
            pltpu.VMEM((HALF,), jnp.float32),
            pltpu.VMEM((HALF,), jnp.float32),
            pltpu.VMEM((HALF,), jnp.float32),
            pltpu.VMEM((HALF,), jnp.float32),
            pltpu.SemaphoreType.DMA,
        ],
        compiler_params=pltpu.CompilerParams(use_tc_tiling_on_sc=False,
                                             needs_layout_passes=False),
    )(W_t, W_l, W_u, traj_flat, mat_flat)


BB = 32


def _tc_delta_body(len_ref, dse_ref, dte_ref, dso_ref, dto_ref,
                   wsu_ref, wsl_ref, wtu_ref, wtl_ref, out_ref):
    wsl = wsl_ref[:, :]
    wsu = wsu_ref[:, :]
    wtl = wtl_ref[:, :]
    wtu = wtu_ref[:, :]
    bas = jnp.concatenate([wsl + wtl, wsl + wtl], axis=1)
    sv = jnp.concatenate([wsu - wsl, wsu - wsl], axis=1)
    tv = jnp.concatenate([wtu - wtl, wtu - wtl], axis=1)
    dbas = bas[1] - bas[0]
    dsv = sv[1] - sv[0]
    dtv = tv[1] - tv[0]

    q = lax.broadcasted_iota(jnp.int32, (BB, Q), 1)
    pe = 2 * q
    po = 2 * q + 1
    ie = pe // L
    je = pe - ie * L
    io = po // L
    jo = po - io * L
    ln = len_ref[:, :]
    me = ((ie < ln) & (je < ln)).astype(jnp.float32)[:, :, None]
    mo = ((io < ln) & (jo < ln)).astype(jnp.float32)[:, :, None]

    lane = lax.broadcasted_iota(jnp.int32, (BB, Q, 2 * D), 2)
    even = lane < D
    m = jnp.where(even, me, mo)
    ds = jnp.where(even, dse_ref[:, :][:, :, None], dso_ref[:, :][:, :, None])
    dt = jnp.where(even, dte_ref[:, :][:, :, None], dto_ref[:, :][:, :, None])

    u = bas[0] + ds * sv[0] + dt * tv[0]
    v = dbas + ds * dsv + dt * dtv
    out_ref[:, :, :] = u + m * v


def _tc_delta(len2, dse, dte, dso, dto, W_su, W_sl, W_tu, W_tl):
    grid = (B // BB,)
    stream_spec = pl.BlockSpec((BB, Q), lambda b: (b, 0))
    table_spec = pl.BlockSpec((2, D), lambda b: (0, 0))
    return pl.pallas_call(
        _tc_delta_body,
        grid=grid,
        in_specs=[pl.BlockSpec((BB, 1), lambda b: (b, 0)),
                  stream_spec, stream_spec, stream_spec, stream_spec,
                  table_spec, table_spec, table_spec, table_spec],
        out_specs=pl.BlockSpec((BB, Q, 2 * D), lambda b: (b, 0, 0)),
        out_shape=jax.ShapeDtypeStruct((B, Q, 2 * D), jnp.float32),
        compiler_params=pltpu.CompilerParams(
            dimension_semantics=("arbitrary",)),
    )(len2, dse, dte, dso, dto, W_su, W_sl, W_tu, W_tl)


def kernel(traj_input, mat_input, traj_length, W_t, W_l, W_u, W_su, W_sl,
           W_tu, W_tl):
    traj_flat = traj_input.reshape(ROWS * 3)
    mat_flat = mat_input.reshape(B * L * L * 2)

    joint, dse, dte, dso, dto = _sc_joint_and_streams(
        W_t, W_l, W_u, traj_flat, mat_flat)

    delta = _tc_delta(traj_length.reshape(B, 1), dse.reshape(B, Q),
                      dte.reshape(B, Q), dso.reshape(B, Q), dto.reshape(B, Q),
                      W_su, W_sl, W_tu, W_tl)
    return (joint.reshape(B, L, D), delta.reshape(B, L, L, D))


# speedup vs baseline: 3.1989x; 3.1989x over previous
import jax
import jax.numpy as jnp
from jax import lax
from jax.experimental import pallas as pl
from jax.experimental.pallas import tpu as pltpu
from jax.experimental.pallas import tpu_sc as plsc

B, L, D = 1024, 20, 64
HOURS = 24 * 7
NC, NS = 2, 16
NW = NC * NS
ROWS = B * L
RPW = ROWS // NW
GCHUNK = 128
NCHUNK = RPW // GCHUNK


def _sc_body(wt_hbm, wl_hbm, wu_hbm, traj_hbm, out_hbm,
             traj_v, uidx_v, lidx_v, tidx_v, acc_v, tmp_v, sem):
    wid = lax.axis_index("s") * NC + lax.axis_index("c")
    base = wid * RPW
    pltpu.sync_copy(traj_hbm.at[pl.ds(base * 3, RPW * 3)], traj_v)

    def split_idx(k, _):
        off = 48 * k + 3 * lax.iota(jnp.int32, 16)
        sl = pl.ds(k * 16, 16)
        uidx_v[sl] = plsc.load_gather(traj_v, [off])
        lidx_v[sl] = plsc.load_gather(traj_v, [off + 1])
        t = plsc.load_gather(traj_v, [off + 2])
        tidx_v[sl] = (t + (HOURS - 1)) % HOURS + 1
        return 0

    lax.fori_loop(0, RPW // 16, split_idx, 0, unroll=4)

    cps = []
    for idx_v, table, dst_v in ((uidx_v, wu_hbm, acc_v),
                                (lidx_v, wl_hbm, tmp_v)):
        for k in range(NCHUNK):
            cps.append(pltpu.async_copy(
                table.at[idx_v.at[pl.ds(k * GCHUNK, GCHUNK)]],
                dst_v.at[pl.ds(k * GCHUNK, GCHUNK), :], sem))
    for cp in cps:
        cp.wait()

    def addrow(r, _):
        for c in range(D // 16):
            sl = pl.ds(c * 16, 16)
            acc_v[r, sl] = acc_v[r, sl] + tmp_v[r, sl]
        return 0

    lax.fori_loop(0, RPW, addrow, 0, unroll=4)

    cps = []
    for k in range(NCHUNK):
        cps.append(pltpu.async_copy(
            wt_hbm.at[tidx_v.at[pl.ds(k * GCHUNK, GCHUNK)]],
            tmp_v.at[pl.ds(k * GCHUNK, GCHUNK), :], sem))
    for cp in cps:
        cp.wait()
    lax.fori_loop(0, RPW, addrow, 0, unroll=4)
    pltpu.sync_copy(acc_v, out_hbm.at[pl.ds(base, RPW)])


def _sc_joint(W_t, W_l, W_u, traj_flat):
    mesh = plsc.VectorSubcoreMesh(core_axis_name="c", subcore_axis_name="s")
    return pl.kernel(
        _sc_body,
        out_type=jax.ShapeDtypeStruct((ROWS, D), jnp.float32),
        mesh=mesh,
        scratch_types=[
            pltpu.VMEM((RPW * 3,), jnp.int32),
            pltpu.VMEM((RPW,), jnp.int32),
            pltpu.VMEM((RPW,), jnp.int32),
            pltpu.VMEM((RPW,), jnp.int32),
            pltpu.VMEM((RPW, D), jnp.float32),
            pltpu.VMEM((RPW, D), jnp.float32),
            pltpu.SemaphoreType.DMA,
        ],
        compiler_params=pltpu.CompilerParams(use_tc_tiling_on_sc=False,
                                             needs_layout_passes=False),
    )(W_t, W_l, W_u, traj_flat)


PB = 40
NB = 256


def _tc_delta_body(len_ref, x_ref, wsu_ref, wsl_ref, wtu_ref, wtl_ref,
                   out_ref):
    wsl = wsl_ref[:, :]
    wsu = wsu_ref[:, :]
    wtl = wtl_ref[:, :]
    wtu = wtu_ref[:, :]
    bas = wsl + wtl
    sv = wsu - wsl
    tv = wtu - wtl
    bas0 = jnp.broadcast_to(bas[:, 0:1], (D, NB))
    bas1 = jnp.broadcast_to(bas[:, 1:2], (D, NB))
    sv0 = jnp.broadcast_to(sv[:, 0:1], (D, NB))
    sv1 = jnp.broadcast_to(sv[:, 1:2], (D, NB))
    tv0 = jnp.broadcast_to(tv[:, 0:1], (D, NB))
    tv1 = jnp.broadcast_to(tv[:, 1:2], (D, NB))

    ln = len_ref[:, :]
    pid = pl.program_id(0)
    for pi in range(PB):
        pg = pid * PB + pi
        i = pg // L
        j = pg - i * L
        m = (i < ln) & (j < ln)
        ds = x_ref[2 * pi:2 * pi + 1, :]
        dt = x_ref[2 * pi + 1:2 * pi + 2, :]
        a = jnp.where(m, bas1, bas0)
        s = jnp.where(m, sv1, sv0)
        t = jnp.where(m, tv1, tv0)
        out_ref[pi, :, :] = a + ds * s + dt * t


def _tc_delta(len2, xt, wsuT, wslT, wtuT, wtlT):
    grid = (L * L // PB, B // NB)
    table_spec = pl.BlockSpec((D, 2), lambda p, b: (0, 0))
    return pl.pallas_call(
        _tc_delta_body,
        grid=grid,
        in_specs=[pl.BlockSpec((1, NB), lambda p, b: (0, b)),
                  pl.BlockSpec((2 * PB, NB), lambda p, b: (p, b)),
                  table_spec, table_spec, table_spec, table_spec],
        out_specs=pl.BlockSpec((PB, D, NB), lambda p, b: (p, 0, b)),
        out_shape=jax.ShapeDtypeStruct((L * L, D, B), jnp.float32),
        compiler_params=pltpu.CompilerParams(
            dimension_semantics=("arbitrary", "arbitrary")),
    )(len2, xt, wsuT, wslT, wtuT, wtlT)


def kernel(traj_input, mat_input, traj_length, W_t, W_l, W_u, W_su, W_sl,
           W_tu, W_tl):
    traj_flat = traj_input.reshape(ROWS * 3)
    joint = _sc_joint(W_t, W_l, W_u, traj_flat).reshape(B, L, D)

    xt = jnp.transpose(mat_input, (1, 2, 3, 0)).reshape(L * L * 2, B)
    out_t = _tc_delta(traj_length.reshape(1, B), xt,
                      W_su.T, W_sl.T, W_tu.T, W_tl.T)
    delta = jnp.transpose(out_t.reshape(L, L, D, B), (3, 0, 1, 2))
    return (joint, delta)


# speedup vs baseline: 3.2012x; 1.0007x over previous
import jax
import jax.numpy as jnp
from jax import lax
from jax.experimental import pallas as pl
from jax.experimental.pallas import tpu as pltpu
from jax.experimental.pallas import tpu_sc as plsc

B, L, D = 1024, 20, 64
HOURS = 24 * 7
NC, NS = 2, 16
NW = NC * NS
ROWS = B * L
RPW = ROWS // NW
GCHUNK = 128
NCHUNK = RPW // GCHUNK


def _sc_body(wt_hbm, wl_hbm, wu_hbm, traj_hbm, out_hbm,
             traj_v, uidx_v, lidx_v, tidx_v, acc_v, tmp_v, sem):
    wid = lax.axis_index("s") * NC + lax.axis_index("c")
    base = wid * RPW
    pltpu.sync_copy(traj_hbm.at[pl.ds(base * 3, RPW * 3)], traj_v)

    def split_idx(k, _):
        off = 48 * k + 3 * lax.iota(jnp.int32, 16)
        sl = pl.ds(k * 16, 16)
        uidx_v[sl] = plsc.load_gather(traj_v, [off])
        lidx_v[sl] = plsc.load_gather(traj_v, [off + 1])
        t = plsc.load_gather(traj_v, [off + 2])
        tidx_v[sl] = (t + (HOURS - 1)) % HOURS + 1
        return 0

    lax.fori_loop(0, RPW // 16, split_idx, 0, unroll=4)

    cps = []
    for idx_v, table, dst_v in ((uidx_v, wu_hbm, acc_v),
                                (lidx_v, wl_hbm, tmp_v)):
        for k in range(NCHUNK):
            cps.append(pltpu.async_copy(
                table.at[idx_v.at[pl.ds(k * GCHUNK, GCHUNK)]],
                dst_v.at[pl.ds(k * GCHUNK, GCHUNK), :], sem))
    for cp in cps:
        cp.wait()

    def addrow(r, _):
        for c in range(D // 16):
            sl = pl.ds(c * 16, 16)
            acc_v[r, sl] = acc_v[r, sl] + tmp_v[r, sl]
        return 0

    lax.fori_loop(0, RPW, addrow, 0, unroll=4)

    cps = []
    for k in range(NCHUNK):
        cps.append(pltpu.async_copy(
            wt_hbm.at[tidx_v.at[pl.ds(k * GCHUNK, GCHUNK)]],
            tmp_v.at[pl.ds(k * GCHUNK, GCHUNK), :], sem))
    for cp in cps:
        cp.wait()
    lax.fori_loop(0, RPW, addrow, 0, unroll=4)
    pltpu.sync_copy(acc_v, out_hbm.at[pl.ds(base, RPW)])


def _sc_joint(W_t, W_l, W_u, traj_flat):
    mesh = plsc.VectorSubcoreMesh(core_axis_name="c", subcore_axis_name="s")
    return pl.kernel(
        _sc_body,
        out_type=jax.ShapeDtypeStruct((ROWS, D), jnp.float32),
        mesh=mesh,
        scratch_types=[
            pltpu.VMEM((RPW * 3,), jnp.int32),
            pltpu.VMEM((RPW,), jnp.int32),
            pltpu.VMEM((RPW,), jnp.int32),
            pltpu.VMEM((RPW,), jnp.int32),
            pltpu.VMEM((RPW, D), jnp.float32),
            pltpu.VMEM((RPW, D), jnp.float32),
            pltpu.SemaphoreType.DMA,
        ],
        compiler_params=pltpu.CompilerParams(use_tc_tiling_on_sc=False,
                                             needs_layout_passes=False),
    )(W_t, W_l, W_u, traj_flat)


PB = 40
NB = 256


def _tc_delta_body(len_ref, x_ref, wsu_ref, wsl_ref, wtu_ref, wtl_ref,
                   out_ref):
    wsl = wsl_ref[:, :]
    wsu = wsu_ref[:, :]
    wtl = wtl_ref[:, :]
    wtu = wtu_ref[:, :]
    bas = wsl + wtl
    sv = wsu - wsl
    tv = wtu - wtl
    bas0 = jnp.broadcast_to(bas[:, 0:1], (D, NB))
    bas1 = jnp.broadcast_to(bas[:, 1:2], (D, NB))
    sv0 = jnp.broadcast_to(sv[:, 0:1], (D, NB))
    sv1 = jnp.broadcast_to(sv[:, 1:2], (D, NB))
    tv0 = jnp.broadcast_to(tv[:, 0:1], (D, NB))
    tv1 = jnp.broadcast_to(tv[:, 1:2], (D, NB))

    ln = len_ref[:, :]
    pid = pl.program_id(0)
    for pi in range(PB):
        pg = pid * PB + pi
        i = pg // L
        j = pg - i * L
        m = (i < ln) & (j < ln)
        ds = x_ref[2 * pi:2 * pi + 1, :]
        dt = x_ref[2 * pi + 1:2 * pi + 2, :]
        a = jnp.where(m, bas1, bas0)
        s = jnp.where(m, sv1, sv0)
        t = jnp.where(m, tv1, tv0)
        out_ref[pi, :, :] = a + ds * s + dt * t


def _tc_delta(len2, xt, wsuT, wslT, wtuT, wtlT):
    grid = (L * L // PB, B // NB)
    table_spec = pl.BlockSpec((D, 2), lambda p, b: (0, 0))
    return pl.pallas_call(
        _tc_delta_body,
        grid=grid,
        in_specs=[pl.BlockSpec((1, NB), lambda p, b: (0, b)),
                  pl.BlockSpec((2 * PB, NB), lambda p, b: (p, b)),
                  table_spec, table_spec, table_spec, table_spec],
        out_specs=pl.BlockSpec((PB, D, NB), lambda p, b: (p, 0, b)),
        out_shape=jax.ShapeDtypeStruct((L * L, D, B), jnp.float32),
        compiler_params=pltpu.CompilerParams(
            dimension_semantics=("arbitrary", "arbitrary")),
    )(len2, xt, wsuT, wslT, wtuT, wtlT)


def kernel(traj_input, mat_input, traj_length, W_t, W_l, W_u, W_su, W_sl,
           W_tu, W_tl):
    xt = jnp.transpose(mat_input, (1, 2, 3, 0)).reshape(L * L * 2, B)
    out_t = _tc_delta(traj_length.reshape(1, B), xt,
                      W_su.T, W_sl.T, W_tu.T, W_tl.T)
    delta = jnp.transpose(out_t.reshape(L, L, D, B), (3, 0, 1, 2))

    traj_flat = traj_input.reshape(ROWS * 3)
    joint = _sc_joint(W_t, W_l, W_u, traj_flat).reshape(B, L, D)
    return (joint, delta)
